# Initial kernel scaffold; baseline (speedup 1.0000x reference)
import jax
import jax.numpy as jnp
from jax.experimental import pallas as pl


def kernel(x, edge_index, edge_feature, node_states, W_msg, W_skip, b_skip, conv_bias, W_z, b_z, W_r, b_r, W_h, b_h):
    raise NotImplementedError("write your pallas kernel here")


# speedup vs baseline: 2.4976x; 2.4976x over previous
import functools

import jax
import jax.numpy as jnp
from jax import lax
from jax.experimental import pallas as pl
from jax.experimental.pallas import tpu as pltpu
from jax.experimental.pallas import tpu_sc as plsc

N = 10000
E = 320000
D_IN = 128
D_OUT = 128
D_EDGE = 16

N_PAD = 10240
CHUNK = 128
NCH = 2528
CPW = NCH // 32
E_PAD = NCH * CHUNK
ROWS_PER_TILE = N_PAD // 16
_WB = ROWS_PER_TILE // CHUNK


def _dot_t(a, w):
    return lax.dot_general(a, w, (((1,), (1,)), ((), ())),
                           preferred_element_type=jnp.float32)


_BN = 1000
_BCH = 32


def _xw_body(x_ref, wmsg, o_ref):
    o_ref[...] = _dot_t(x_ref[...], wmsg[:, :D_IN])


def _xw(x, W_msg):
    return pl.pallas_call(
        _xw_body,
        grid=(N // _BN,),
        in_specs=[
            pl.BlockSpec((_BN, D_IN), lambda i: (i, 0)),
            pl.BlockSpec((D_OUT, D_IN + D_EDGE), lambda i: (0, 0)),
        ],
        out_specs=pl.BlockSpec((_BN, D_OUT), lambda i: (i, 0)),
        out_shape=jax.ShapeDtypeStruct((N, D_OUT), jnp.float32),
    )(x, W_msg)


def _efw_body(ef_ref, wmsg, o_ref):
    ef = ef_ref[...].reshape(_BCH * CHUNK, D_EDGE)
    o_ref[...] = _dot_t(ef, wmsg[:, D_IN:]).reshape(_BCH, CHUNK, D_OUT)


def _efw(ef3d, W_msg):
    return pl.pallas_call(
        _efw_body,
        grid=(NCH // _BCH,),
        in_specs=[
            pl.BlockSpec((_BCH, CHUNK, D_EDGE), lambda i: (i, 0, 0)),
            pl.BlockSpec((D_OUT, D_IN + D_EDGE), lambda i: (0, 0)),
        ],
        out_specs=pl.BlockSpec((_BCH, CHUNK, D_OUT), lambda i: (i, 0, 0)),
        out_shape=jax.ShapeDtypeStruct((NCH, CHUNK, D_OUT), jnp.float32),
    )(ef3d, W_msg)


def _sc_body(xw_hbm, efw_hbm, src_hbm, dst_hbm, zz_hbm,
             p_out,
             acc_sh, src_v, dst_v, xrow_v, efrow_v, sem):
    cid = lax.axis_index("c")
    sid = lax.axis_index("s")
    wid = sid * 2 + cid
    row0 = sid * ROWS_PER_TILE

    pltpu.sync_copy(zz_hbm, xrow_v)

    def zbody(r, carry):
        pltpu.sync_copy(xrow_v, acc_sh.at[pl.ds(row0 + r * CHUNK, CHUNK)])
        return carry

    lax.fori_loop(0, _WB, zbody, 0)
    plsc.subcore_barrier()

    base = wid * CPW

    def body(j, carry):
        ch = base + j
        pltpu.sync_copy(src_hbm.at[ch], src_v)
        pltpu.sync_copy(dst_hbm.at[ch], dst_v)
        pltpu.async_copy(xw_hbm.at[src_v], xrow_v, sem).wait()
        pltpu.sync_copy(xrow_v, acc_sh.at[dst_v], add=True)
        pltpu.sync_copy(efw_hbm.at[ch], efrow_v)
        pltpu.sync_copy(efrow_v, acc_sh.at[dst_v], add=True)
        return carry

    lax.fori_loop(0, CPW, body, 0)
    plsc.subcore_barrier()

    def obody(r, carry):
        rr = row0 + r * CHUNK
        pltpu.sync_copy(acc_sh.at[pl.ds(rr, CHUNK)], xrow_v)
        pltpu.sync_copy(xrow_v, p_out.at[cid, pl.ds(rr, CHUNK)])
        return carry

    lax.fori_loop(0, _WB, obody, 0)


@functools.cache
def _sc_aggregate():
    return pl.kernel(
        _sc_body,
        out_type=jax.ShapeDtypeStruct((2, N_PAD, D_OUT), jnp.float32),
        mesh=plsc.VectorSubcoreMesh(core_axis_name="c", subcore_axis_name="s"),
        scratch_types=[
            pltpu.VMEM_SHARED((N_PAD, D_OUT), jnp.float32),
            pltpu.VMEM((CHUNK,), jnp.int32),
            pltpu.VMEM((CHUNK,), jnp.int32),
            pltpu.VMEM((CHUNK, D_OUT), jnp.float32),
            pltpu.VMEM((CHUNK, D_OUT), jnp.float32),
            pltpu.SemaphoreType.DMA,
        ],
    )


def _dense_body(pp, x_ref, hp_ref, wskip, bskip, convb,
                wz, bz, wr, br, wh, bh, out_ref):
    aggr = pp[0] + pp[1]
    x = x_ref[...]
    hp = hp_ref[...]

    h = jnp.maximum(aggr + convb[...] + _dot_t(x, wskip[...]) + bskip[...],
                    0.0)

    w_z = wz[...]
    w_r = wr[...]
    w_h = wh[...]
    z = jax.nn.sigmoid(_dot_t(h, w_z[:, :D_OUT]) + _dot_t(hp, w_z[:, D_OUT:])
                       + bz[...])
    r = jax.nn.sigmoid(_dot_t(h, w_r[:, :D_OUT]) + _dot_t(hp, w_r[:, D_OUT:])
                       + br[...])
    h_tilde = jnp.tanh(_dot_t(h, w_h[:, :D_OUT])
                       + _dot_t(r * hp, w_h[:, D_OUT:]) + bh[...])
    out_ref[...] = z * hp + (1.0 - z) * h_tilde


def _dense(p_par, x, hp, W_skip, b_skip, conv_bias,
           W_z, b_z, W_r, b_r, W_h, b_h):
    full = lambda shape: pl.BlockSpec(shape, lambda i: (0,) * len(shape))
    return pl.pallas_call(
        _dense_body,
        grid=(N // _BN,),
        in_specs=[
            pl.BlockSpec((2, _BN, D_OUT), lambda i: (0, i, 0)),
            pl.BlockSpec((_BN, D_IN), lambda i: (i, 0)),
            pl.BlockSpec((_BN, D_OUT), lambda i: (i, 0)),
            full((D_OUT, D_IN)),
            full((1, D_OUT)),
            full((1, D_OUT)),
            full((D_OUT, D_IN + D_OUT)),
            full((1, D_OUT)),
            full((D_OUT, D_IN + D_OUT)),
            full((1, D_OUT)),
            full((D_OUT, D_IN + D_OUT)),
            full((1, D_OUT)),
        ],
        out_specs=pl.BlockSpec((_BN, D_OUT), lambda i: (i, 0)),
        out_shape=jax.ShapeDtypeStruct((N, D_OUT), jnp.float32),
    )(p_par, x, hp, W_skip, b_skip, conv_bias,
      W_z, b_z, W_r, b_r, W_h, b_h)


def kernel(x, edge_index, edge_feature, node_states,
           W_msg, W_skip, b_skip, conv_bias,
           W_z, b_z, W_r, b_r, W_h, b_h):
    pad = E_PAD - E
    src = jnp.concatenate([edge_index[0], jnp.zeros((pad,), jnp.int32)])
    dst = jnp.concatenate([edge_index[1], jnp.full((pad,), N, jnp.int32)])
    ef = jnp.concatenate(
        [edge_feature, jnp.zeros((pad, D_EDGE), jnp.float32)])
    src2d = src.reshape(NCH, CHUNK)
    dst2d = dst.reshape(NCH, CHUNK)
    ef3d = ef.reshape(NCH, CHUNK, D_EDGE)
    zz = jnp.zeros((CHUNK, D_OUT), jnp.float32)

    xw = _xw(x, W_msg)
    efw = _efw(ef3d, W_msg)
    p_par = _sc_aggregate()(xw, efw, src2d, dst2d, zz)

    return _dense(p_par, x, node_states, W_skip,
                  b_skip.reshape(1, -1), conv_bias.reshape(1, -1),
                  W_z, b_z.reshape(1, -1), W_r, b_r.reshape(1, -1),
                  W_h, b_h.reshape(1, -1))
